# Initial kernel scaffold; baseline (speedup 1.0000x reference)
import functools

import jax
import jax.numpy as jnp
from jax import lax
from jax.experimental import pallas as pl
from jax.experimental.pallas import tpu as pltpu
from jax.experimental.pallas import tpu_sc as plsc


CHUNK = 80


def _make_agg(n_nodes, n_edges, d):
    info = plsc.get_sparse_core_info()
    nc, ns = info.num_cores, info.num_subcores
    rows_per_core = n_nodes // nc
    edges_per_tile = n_edges // ns
    n_chunks = edges_per_tile // CHUNK
    rows_per_tile = rows_per_core // ns
    rows_rem = rows_per_core - rows_per_tile * ns

    mesh = plsc.VectorSubcoreMesh(core_axis_name="c", subcore_axis_name="s")

    @functools.partial(
        pl.kernel,
        out_type=jax.ShapeDtypeStruct((n_nodes, d), jnp.float32),
        mesh=mesh,
        scratch_types=[
            pltpu.VMEM((CHUNK,), jnp.int32),
            pltpu.VMEM((CHUNK,), jnp.int32),
            pltpu.VMEM((CHUNK, d), jnp.float32),
            pltpu.VMEM_SHARED((rows_per_core + 1, d), jnp.float32),
            pltpu.SemaphoreType.DMA,
        ],
    )
    def agg(x_hbm, src_hbm, dst_hbm, out_hbm, src_v, dst_v, rows_v, acc, sem):
        cid = lax.axis_index("c")
        sid = lax.axis_index("s")
        base = cid * rows_per_core

        r0 = sid * rows_per_tile
        pltpu.sync_copy(x_hbm.at[pl.ds(base + r0, rows_per_tile)],
                        acc.at[pl.ds(r0, rows_per_tile)])

        @pl.when(sid == 0)
        def _():
            if rows_rem:
                rr = rows_per_tile * ns
                pltpu.sync_copy(x_hbm.at[pl.ds(base + rr, rows_rem)],
                                acc.at[pl.ds(rr, rows_rem)])

        plsc.subcore_barrier()

        e0 = sid * edges_per_tile

        def body(i, carry):
            cb = e0 + i * CHUNK
            pltpu.sync_copy(src_hbm.at[pl.ds(cb, CHUNK)], src_v)
            pltpu.sync_copy(dst_hbm.at[pl.ds(cb, CHUNK)], dst_v)
            for j in range(CHUNK // 16):
                dvec = dst_v[pl.ds(j * 16, 16)]
                r = dvec - base
                ok = (r >= 0) & (r < rows_per_core)
                dst_v[pl.ds(j * 16, 16)] = jnp.where(ok, r, rows_per_core)
            pltpu.async_copy(x_hbm.at[src_v], rows_v, sem).wait()
            pltpu.sync_copy(rows_v, acc.at[dst_v], add=True)
            return carry

        lax.fori_loop(0, n_chunks, body, 0)

        plsc.subcore_barrier()

        pltpu.sync_copy(acc.at[pl.ds(r0, rows_per_tile)],
                        out_hbm.at[pl.ds(base + r0, rows_per_tile)])

        @pl.when(sid == 0)
        def _():
            if rows_rem:
                rr = rows_per_tile * ns
                pltpu.sync_copy(acc.at[pl.ds(rr, rows_rem)],
                                out_hbm.at[pl.ds(base + rr, rows_rem)])

    return agg


def _matmul_body(a_ref, w_ref, o_ref):
    o_ref[...] = jnp.dot(a_ref[...], w_ref[...],
                         preferred_element_type=jnp.float32)


def _matmul(agg, weight):
    n, d_in = agg.shape
    d_out = weight.shape[1]
    blk = 2000
    return pl.pallas_call(
        _matmul_body,
        grid=(n // blk,),
        in_specs=[
            pl.BlockSpec((blk, d_in), lambda i: (i, 0)),
            pl.BlockSpec((d_in, d_out), lambda i: (0, 0)),
        ],
        out_specs=pl.BlockSpec((blk, d_out), lambda i: (i, 0)),
        out_shape=jax.ShapeDtypeStruct((n, d_out), jnp.float32),
    )(agg, weight)


def kernel(x, weight, edge_index):
    n_nodes, d = x.shape
    n_edges = edge_index.shape[1]
    agg_fn = _make_agg(n_nodes, n_edges, d)
    agg = agg_fn(x, edge_index[0], edge_index[1])
    return _matmul(agg, weight)


# speedup vs baseline: 1.6105x; 1.6105x over previous
import functools

import jax
import jax.numpy as jnp
from jax import lax
from jax.experimental import pallas as pl
from jax.experimental.pallas import tpu as pltpu
from jax.experimental.pallas import tpu_sc as plsc


SCAN_C = 1600
GB = 64
CAP = 8320


def _make_agg(n_pad, n_edges, d, rpt, nw):
    n_scan = n_edges // SCAN_C
    trash = CAP - 16

    mesh = plsc.VectorSubcoreMesh(core_axis_name="c", subcore_axis_name="s")

    @functools.partial(
        pl.kernel,
        out_type=jax.ShapeDtypeStruct((n_pad, d), jnp.float32),
        mesh=mesh,
        compiler_params=pltpu.CompilerParams(needs_layout_passes=False),
        scratch_types=[
            pltpu.VMEM((SCAN_C,), jnp.int32),
            pltpu.VMEM((SCAN_C,), jnp.int32),
            pltpu.VMEM((CAP,), jnp.int32),
            pltpu.VMEM((CAP,), jnp.int32),
            pltpu.VMEM((GB, d), jnp.float32),
            pltpu.VMEM((rpt + 1, d), jnp.float32),
            pltpu.SemaphoreType.DMA,
        ],
    )
    def agg(x_hbm, src_hbm, dst_hbm, out_hbm,
            src_ch, dst_ch, csrc, cdst, rows_v, acc_v, sem):
        cid = lax.axis_index("c")
        sid = lax.axis_index("s")
        wid = cid * 16 + sid
        base = wid * rpt
        lane = lax.iota(jnp.int32, 16)

        pltpu.sync_copy(x_hbm.at[pl.ds(base, rpt)], acc_v.at[pl.ds(0, rpt)])

        def process_batch(rd):
            rd = pl.multiple_of(rd, 8)
            pltpu.async_copy(x_hbm.at[csrc.at[pl.ds(rd, GB)]], rows_v,
                             sem).wait()

            def rmw(e16, carry):
                v = cdst[pl.ds(pl.multiple_of(rd + e16 * 16, 8), 16)]
                for ln in range(16):
                    row = v[ln]
                    e = e16 * 16 + ln
                    for c in range(d // 16):
                        sl = pl.ds(c * 16, 16)
                        acc_v[row, sl] = acc_v[row, sl] + rows_v[e, sl]
                return carry

            lax.fori_loop(0, GB // 16, rmw, 0)

        def scan_step(s, carry):
            cnt, rd = carry
            eb = s * SCAN_C
            pltpu.sync_copy(src_hbm.at[pl.ds(eb, SCAN_C)], src_ch)
            pltpu.sync_copy(dst_hbm.at[pl.ds(eb, SCAN_C)], dst_ch)

            def compact(j, cnt):
                dl = dst_ch[pl.ds(j * 16, 16)] - base
                sv = src_ch[pl.ds(j * 16, 16)]
                m = (dl >= 0) & (dl < rpt)
                mi = jnp.where(m, jnp.int32(1), jnp.int32(0))
                pref = plsc.cumsum(mi)
                pos = jnp.where(m, cnt + pref - mi, trash + lane)
                plsc.store_scatter(csrc, [pos], sv)
                plsc.store_scatter(cdst, [pos], dl)
                return cnt + pref[15]

            cnt = lax.fori_loop(0, SCAN_C // 16, compact, cnt)

            def drain_cond(c):
                return c[0] - c[1] >= GB

            def drain(c):
                cnt, rd = c
                process_batch(rd)
                return (cnt, rd + GB)

            return lax.while_loop(drain_cond, drain, (cnt, rd))

        cnt, rd = lax.fori_loop(0, n_scan, scan_step, (jnp.int32(0), jnp.int32(0)))

        for j in range(GB // 16):
            pos = cnt + j * 16 + lane
            plsc.store_scatter(csrc, [pos], base + lane)
            plsc.store_scatter(cdst, [pos], jnp.full((16,), rpt, jnp.int32))
        process_batch(rd)

        pltpu.sync_copy(acc_v.at[pl.ds(0, rpt)], out_hbm.at[pl.ds(base, rpt)])

    return agg


def _matmul_body(a_ref, w_ref, o_ref):
    o_ref[...] = jnp.dot(a_ref[...], w_ref[...],
                         preferred_element_type=jnp.float32)


def _matmul(agg, weight):
    n, d_in = agg.shape
    d_out = weight.shape[1]
    blk = 2000
    return pl.pallas_call(
        _matmul_body,
        grid=(n // blk,),
        in_specs=[
            pl.BlockSpec((blk, d_in), lambda i: (i, 0)),
            pl.BlockSpec((d_in, d_out), lambda i: (0, 0)),
        ],
        out_specs=pl.BlockSpec((blk, d_out), lambda i: (i, 0)),
        out_shape=jax.ShapeDtypeStruct((n, d_out), jnp.float32),
    )(agg, weight)


def kernel(x, weight, edge_index):
    n_nodes, d = x.shape
    n_edges = edge_index.shape[1]
    nw = 32
    rpt = (-(-n_nodes // nw) + 7) // 8 * 8
    n_pad = rpt * nw
    x_pad = jnp.zeros((n_pad, d), x.dtype).at[:n_nodes].set(x)
    agg_fn = _make_agg(n_pad, n_edges, d, rpt, nw)
    agg = agg_fn(x_pad, edge_index[0], edge_index[1])
    return _matmul(agg[:n_nodes], weight)


# speedup vs baseline: 3.1208x; 1.9377x over previous
import functools

import jax
import jax.numpy as jnp
from jax import lax
from jax.experimental import pallas as pl
from jax.experimental.pallas import tpu as pltpu
from jax.experimental.pallas import tpu_sc as plsc


SCAN_C = 1600
GB = 48
CAP = 6448


def _make_agg(n_nodes, n_edges, d, rpt, nw):
    n_scan = n_edges // SCAN_C
    trash = CAP - 16
    last = n_nodes - (nw - 1) * rpt

    mesh = plsc.VectorSubcoreMesh(core_axis_name="c", subcore_axis_name="s")

    @functools.partial(
        pl.kernel,
        out_type=jax.ShapeDtypeStruct((n_nodes, d), jnp.float32),
        mesh=mesh,
        compiler_params=pltpu.CompilerParams(needs_layout_passes=False),
        scratch_types=[
            pltpu.VMEM((SCAN_C,), jnp.int32),
            pltpu.VMEM((SCAN_C,), jnp.int32),
            pltpu.VMEM((SCAN_C,), jnp.int32),
            pltpu.VMEM((SCAN_C,), jnp.int32),
            pltpu.VMEM((CAP,), jnp.int32),
            pltpu.VMEM((CAP,), jnp.int32),
            pltpu.VMEM((GB, d), jnp.float32),
            pltpu.VMEM((GB, d), jnp.float32),
            pltpu.VMEM((rpt + 1, d), jnp.float32),
            pltpu.SemaphoreType.DMA,
            pltpu.SemaphoreType.DMA,
            pltpu.SemaphoreType.DMA,
            pltpu.SemaphoreType.DMA,
        ],
    )
    def agg(x_hbm, src_hbm, dst_hbm, out_hbm,
            src0, src1, dst0, dst1, csrc, cdst, rows0, rows1, acc_v,
            sem_s0, sem_s1, sem_g0, sem_g1):
        cid = lax.axis_index("c")
        sid = lax.axis_index("s")
        wid = cid * 16 + sid
        base = wid * rpt
        lane = lax.iota(jnp.int32, 16)
        srcs = (src0, src1)
        dsts = (dst0, dst1)
        rows = (rows0, rows1)
        sem_s = (sem_s0, sem_s1)
        sem_g = (sem_g0, sem_g1)

        @pl.when(wid < nw - 1)
        def _():
            pltpu.sync_copy(x_hbm.at[pl.ds(base, rpt)], acc_v.at[pl.ds(0, rpt)])

        @pl.when(wid == nw - 1)
        def _():
            pltpu.sync_copy(x_hbm.at[pl.ds(base, last)],
                            acc_v.at[pl.ds(0, last)])

        def fire_scan(k, b):
            eb = k * SCAN_C
            pltpu.async_copy(src_hbm.at[pl.ds(eb, SCAN_C)], srcs[b], sem_s[b])
            pltpu.async_copy(dst_hbm.at[pl.ds(eb, SCAN_C)], dsts[b], sem_s[b])

        def wait_scan(b):
            pltpu.make_async_copy(src_hbm.at[pl.ds(0, SCAN_C)], srcs[b],
                                  sem_s[b]).wait()
            pltpu.make_async_copy(dst_hbm.at[pl.ds(0, SCAN_C)], dsts[b],
                                  sem_s[b]).wait()

        fire_scan(0, 0)

        def compact_chunk(k, b, cnt):
            wait_scan(b)

            @pl.when(k + 1 < n_scan)
            def _():
                fire_scan(k + 1, 1 - b)

            def compact(j, cnt):
                o = j * 80
                parts = []
                for u in range(5):
                    dl = dsts[b][pl.ds(o + u * 16, 16)] - base
                    sv = srcs[b][pl.ds(o + u * 16, 16)]
                    m = (dl >= 0) & (dl < rpt)
                    mi = jnp.where(m, jnp.int32(1), jnp.int32(0))
                    parts.append((dl, sv, m, mi, plsc.cumsum(mi)))
                for dl, sv, m, mi, pref in parts:
                    pos = jnp.where(m, cnt + pref - mi, trash + lane)
                    plsc.store_scatter(csrc, [pos], sv)
                    plsc.store_scatter(cdst, [pos], dl)
                    cnt = cnt + pref[15]
                return cnt

            return lax.fori_loop(0, SCAN_C // 80, compact, cnt)

        def scan_pair(i, cnt):
            cnt = compact_chunk(2 * i, 0, cnt)
            cnt = compact_chunk(2 * i + 1, 1, cnt)
            return cnt

        cnt = lax.fori_loop(0, n_scan // 2, scan_pair, jnp.int32(0))

        for j in range(4 * GB // 16):
            pos = cnt + j * 16 + lane
            plsc.store_scatter(csrc, [pos], lane + 16 * (j % 4))
            plsc.store_scatter(cdst, [pos], jnp.full((16,), rpt, jnp.int32))

        def fire_gather(k, b):
            rd = pl.multiple_of(k * GB, 8)
            pltpu.async_copy(x_hbm.at[csrc.at[pl.ds(rd, GB)]], rows[b],
                             sem_g[b])

        def wait_gather(b):
            pltpu.make_async_copy(x_hbm.at[pl.ds(0, GB)], rows[b],
                                  sem_g[b]).wait()

        def rmw_batch(k, b):
            rd = k * GB

            def rmw(e16, carry):
                v = cdst[pl.ds(pl.multiple_of(rd + e16 * 16, 16), 16)]
                for ln in range(16):
                    row = v[ln]
                    e = e16 * 16 + ln
                    for c in range(d // 16):
                        sl = pl.ds(c * 16, 16)
                        plsc.addupdate(acc_v.at[row, sl], rows[b][e, sl])
                return carry

            lax.fori_loop(0, GB // 16, rmw, 0)

        nb2 = (cnt + 2 * GB - 1) // (2 * GB)
        fire_gather(0, 0)
        fire_gather(1, 1)

        def proc_pair(i, carry):
            for b in range(2):
                k = 2 * i + b
                wait_gather(b)
                rmw_batch(k, b)
                fire_gather(k + 2, b)
            return carry

        lax.fori_loop(0, nb2, proc_pair, 0)
        wait_gather(0)
        wait_gather(1)

        @pl.when(wid < nw - 1)
        def _():
            pltpu.sync_copy(acc_v.at[pl.ds(0, rpt)],
                            out_hbm.at[pl.ds(base, rpt)])

        @pl.when(wid == nw - 1)
        def _():
            pltpu.sync_copy(acc_v.at[pl.ds(0, last)],
                            out_hbm.at[pl.ds(base, last)])

    return agg


def _matmul_body(a_ref, w_ref, o_ref):
    o_ref[...] = jnp.dot(a_ref[...], w_ref[...],
                         preferred_element_type=jnp.float32)


def _matmul(agg, weight):
    n, d_in = agg.shape
    d_out = weight.shape[1]
    blk = 2000
    return pl.pallas_call(
        _matmul_body,
        grid=(n // blk,),
        in_specs=[
            pl.BlockSpec((blk, d_in), lambda i: (i, 0)),
            pl.BlockSpec((d_in, d_out), lambda i: (0, 0)),
        ],
        out_specs=pl.BlockSpec((blk, d_out), lambda i: (i, 0)),
        out_shape=jax.ShapeDtypeStruct((n, d_out), jnp.float32),
    )(agg, weight)


def kernel(x, weight, edge_index):
    n_nodes, d = x.shape
    n_edges = edge_index.shape[1]
    nw = 32
    rpt = (-(-n_nodes // nw) + 7) // 8 * 8
    agg_fn = _make_agg(n_nodes, n_edges, d, rpt, nw)
    agg = agg_fn(x, edge_index[0], edge_index[1])
    return _matmul(agg, weight)
